# baseline (speedup 1.0000x reference)
import jax
import jax.numpy as jnp
from jax.experimental import pallas as pl

_BB = 2


def _body(ids_ref, x_ref, out_ref):
    w = (ids_ref[:, 0, :] != 0).astype(jnp.float32)
    s = jax.lax.dot_general(
        w, x_ref[...], (((1,), (1,)), ((0,), (0,))),
        preferred_element_type=jnp.float32)
    cnt = jnp.sum(w, axis=1, keepdims=True)
    out_ref[:, 0, :] = s / cnt


def kernel(x, input):
    B, S, D = x.shape
    ids3 = input.reshape(B, 1, S).astype(jnp.int32)
    out = pl.pallas_call(
        _body,
        grid=(B // _BB,),
        in_specs=[
            pl.BlockSpec((_BB, 1, S), lambda b: (b, 0, 0)),
            pl.BlockSpec((_BB, S, D), lambda b: (b, 0, 0)),
        ],
        out_specs=pl.BlockSpec((_BB, 1, D), lambda b: (b, 0, 0)),
        out_shape=jax.ShapeDtypeStruct((B, 1, D), jnp.float32),
    )(ids3, x)
    return out.reshape(B, D)


# speedup vs baseline: 1.0571x; 1.0571x over previous
import jax
import jax.numpy as jnp
from jax.experimental import pallas as pl


def _body(ids_ref, xa_ref, xb_ref, out_ref):
    w = (ids_ref[0] != 0).astype(jnp.float32)
    s2 = xa_ref.shape[1]
    dn = (((1,), (0,)), ((), ()))
    s = (jax.lax.dot_general(w[:, :s2], xa_ref[0], dn,
                             preferred_element_type=jnp.float32)
         + jax.lax.dot_general(w[:, s2:], xb_ref[0], dn,
                               preferred_element_type=jnp.float32))
    cnt = jnp.sum(w)
    out_ref[0] = s / cnt


def kernel(x, input):
    B, S, D = x.shape
    ids3 = input.reshape(B, 1, S).astype(jnp.int32)
    out = pl.pallas_call(
        _body,
        grid=(B,),
        in_specs=[
            pl.BlockSpec((1, 1, S), lambda b: (b, 0, 0)),
            pl.BlockSpec((1, S // 2, D), lambda b: (b, 0, 0)),
            pl.BlockSpec((1, S // 2, D), lambda b: (b, 1, 0)),
        ],
        out_specs=pl.BlockSpec((1, 1, D), lambda b: (b, 0, 0)),
        out_shape=jax.ShapeDtypeStruct((B, 1, D), jnp.float32),
    )(ids3, x, x)
    return out.reshape(B, D)
